# baseline (speedup 1.0000x reference)
import jax
import jax.numpy as jnp
from jax import lax
from jax.experimental import pallas as pl
from jax.experimental.pallas import tpu as pltpu
from jax.experimental.pallas import tpu_sc as plsc

B = 2048
L = 32
D_MODEL = 128
D_LLM = 1024
D_FF = 512
E = 8
D_IN = L * D_MODEL
EPS = 1e-09

BB = 256
N_EG = E * D_MODEL + D_MODEL
N_ALL = D_FF + N_EG

_SC_INFO = plsc.get_sparse_core_info()
NC = _SC_INFO.num_cores
NS = _SC_INFO.num_subcores
NW = NC * NS
TPW = B // NW
SG = TPW // 16
WPB = BB // TPW


def _tc_kernel(x_ref, dkp_ref, cyc_ref, Wgf_ref, Wgd_ref, Wgc_ref, bg_ref,
               Wgo_ref, bgo_ref, We_ref, be_ref, Wgen_ref, bgen_ref,
               lgt_ref, pe_ref, gen_ref, Wall_s, Wgd_s, Wgo_s):
    i = pl.program_id(0)

    @pl.when(i == 0)
    def _cast_weights():
        Wall_s[:, 0:D_FF] = Wgf_ref[...].astype(jnp.bfloat16)
        for e in range(E):
            Wall_s[:, D_FF + e * D_MODEL:D_FF + (e + 1) * D_MODEL] = (
                We_ref[e].astype(jnp.bfloat16))
        Wall_s[:, D_FF + E * D_MODEL:] = Wgen_ref[...].astype(jnp.bfloat16)
        Wgd_s[...] = Wgd_ref[...].astype(jnp.bfloat16)
        Wgo_s[...] = Wgo_ref[...].astype(jnp.bfloat16)

    xb = x_ref[...].reshape(BB, D_IN).astype(jnp.bfloat16)
    dkpb = dkp_ref[...].astype(jnp.bfloat16)

    zg = jnp.dot(xb, Wall_s[:, 0:D_FF], preferred_element_type=jnp.float32)
    zd = jnp.dot(dkpb, Wgd_s[...], preferred_element_type=jnp.float32)
    big = jnp.dot(xb, Wall_s[:, D_FF:], preferred_element_type=jnp.float32)

    z = zg + zd + cyc_ref[...] * Wgc_ref[...] + bg_ref[...]
    hb = jax.nn.gelu(z).astype(jnp.bfloat16)
    logits = jnp.dot(hb, Wgo_s[...], preferred_element_type=jnp.float32) + bgo_ref[...]

    lgt = logits.reshape(WPB, TPW, E).transpose(0, 2, 1)
    lgt_ref[...] = jnp.pad(lgt, ((0, 0), (0, 0), (0, D_MODEL - TPW)))

    for e in range(E):
        pe = big[:, e * D_MODEL:(e + 1) * D_MODEL] + be_ref[e]
        pe_ref[e] = pe.astype(jnp.bfloat16).astype(jnp.float32)

    gen_ref[...] = big[:, E * D_MODEL:] + bgen_ref[...]


def _round_bf16(v):
    c = v * 65537.0
    return c - (c - v)


def _sc_route(lgt_hbm, pe_hbm, gen_hbm, out_hbm,
              lgt_v, gen_v, idx_v, gat_v, rows_v, out_v, sem):
    wid = lax.axis_index("s") * NC + lax.axis_index("c")
    base = wid * TPW
    pltpu.sync_copy(lgt_hbm.at[wid], lgt_v)
    pltpu.sync_copy(gen_hbm.at[pl.ds(base, TPW)], gen_v)

    iota16 = lax.iota(jnp.int32, 16)
    neg_inf = jnp.full((16,), -jnp.inf, jnp.float32)
    for sg in range(SG):
        ls = [lgt_v[e, pl.ds(sg * 16, 16)] for e in range(E)]
        m1 = ls[0]
        a1 = jnp.zeros((16,), jnp.int32)
        for e in range(1, E):
            gt = ls[e] > m1
            m1 = jnp.where(gt, ls[e], m1)
            a1 = jnp.where(gt, jnp.full((16,), e, jnp.int32), a1)
        m2 = neg_inf
        a2 = jnp.zeros((16,), jnp.int32)
        for e in range(E):
            v = jnp.where(a1 == e, neg_inf, ls[e])
            gt = v > m2
            m2 = jnp.where(gt, v, m2)
            a2 = jnp.where(gt, jnp.full((16,), e, jnp.int32), a2)
        s_all = jnp.zeros((16,), jnp.float32)
        for e in range(E):
            s_all = s_all + jnp.exp(ls[e] - m1)
        p2 = jnp.exp(m2 - m1)
        denom = 1.0 + p2 + EPS * s_all
        g1 = _round_bf16(1.0 / denom)
        g2 = _round_bf16(p2 / denom)
        tg = base + sg * 16 + iota16
        idx_v[pl.ds(sg * 16, 16)] = a1 * B + tg
        idx_v[pl.ds(TPW + sg * 16, 16)] = a2 * B + tg
        gat_v[pl.ds(sg * 16, 16)] = g1
        gat_v[pl.ds(TPW + sg * 16, 16)] = g2

    pltpu.async_copy(pe_hbm.at[idx_v], rows_v, sem).wait()

    for sg in range(SG):
        g1blk = gat_v[pl.ds(sg * 16, 16)]
        g2blk = gat_v[pl.ds(TPW + sg * 16, 16)]
        for k in range(16):
            t = sg * 16 + k
            g1s = g1blk[k]
            g2s = g2blk[k]
            for j in range(D_MODEL // 16):
                c = (g1s * rows_v[t, pl.ds(j * 16, 16)]
                     + g2s * rows_v[TPW + t, pl.ds(j * 16, 16)])
                out_v[t, pl.ds(j * 16, 16)] = (
                    _round_bf16(c) + gen_v[t, pl.ds(j * 16, 16)])

    pltpu.sync_copy(out_v, out_hbm.at[pl.ds(base, TPW)])


def kernel(cycle_curve_data, cycle_numbers, DKP_embeddings, Wg_dkp, Wg_cyc,
           Wg_flat, bg, Wg_out, bg_out, We, be, Wgen, bgen):
    b = cycle_curve_data.shape[0]
    bg2 = bg.reshape(1, -1)
    bgo2 = bg_out.reshape(1, -1)
    bgen2 = bgen.reshape(1, -1)

    grid = b // BB
    lgt, pe_all, gen_full = pl.pallas_call(
        _tc_kernel,
        grid=(grid,),
        in_specs=[
            pl.BlockSpec((BB, L, D_MODEL), lambda i: (i, 0, 0)),
            pl.BlockSpec((BB, D_LLM), lambda i: (i, 0)),
            pl.BlockSpec((BB, 1), lambda i: (i, 0)),
            pl.BlockSpec((D_IN, D_FF), lambda i: (0, 0)),
            pl.BlockSpec((D_LLM, D_FF), lambda i: (0, 0)),
            pl.BlockSpec((1, D_FF), lambda i: (0, 0)),
            pl.BlockSpec((1, D_FF), lambda i: (0, 0)),
            pl.BlockSpec((D_FF, E), lambda i: (0, 0)),
            pl.BlockSpec((1, E), lambda i: (0, 0)),
            pl.BlockSpec((E, D_IN, D_MODEL), lambda i: (0, 0, 0)),
            pl.BlockSpec((E, D_MODEL), lambda i: (0, 0)),
            pl.BlockSpec((D_IN, D_MODEL), lambda i: (0, 0)),
            pl.BlockSpec((1, D_MODEL), lambda i: (0, 0)),
        ],
        out_specs=[
            pl.BlockSpec((WPB, E, D_MODEL), lambda i: (i, 0, 0)),
            pl.BlockSpec((E, BB, D_MODEL), lambda i: (0, i, 0)),
            pl.BlockSpec((BB, D_MODEL), lambda i: (i, 0)),
        ],
        out_shape=[
            jax.ShapeDtypeStruct((NW, E, D_MODEL), jnp.float32),
            jax.ShapeDtypeStruct((E, b, D_MODEL), jnp.float32),
            jax.ShapeDtypeStruct((b, D_MODEL), jnp.float32),
        ],
        scratch_shapes=[
            pltpu.VMEM((D_IN, N_ALL), jnp.bfloat16),
            pltpu.VMEM((D_LLM, D_FF), jnp.bfloat16),
            pltpu.VMEM((D_FF, E), jnp.bfloat16),
        ],
        compiler_params=pltpu.CompilerParams(
            dimension_semantics=("arbitrary",),
        ),
    )(cycle_curve_data, DKP_embeddings, cycle_numbers, Wg_flat, Wg_dkp, Wg_cyc,
      bg2, Wg_out, bgo2, We, be, Wgen, bgen2)

    pe_flat = pe_all.reshape(E * b, D_MODEL)

    sc_fn = pl.kernel(
        _sc_route,
        out_type=jax.ShapeDtypeStruct((b, D_MODEL), jnp.float32),
        mesh=plsc.VectorSubcoreMesh(core_axis_name="c", subcore_axis_name="s"),
        scratch_types=[
            pltpu.VMEM((E, D_MODEL), jnp.float32),
            pltpu.VMEM((TPW, D_MODEL), jnp.float32),
            pltpu.VMEM((2 * TPW,), jnp.int32),
            pltpu.VMEM((2 * TPW,), jnp.float32),
            pltpu.VMEM((2 * TPW, D_MODEL), jnp.float32),
            pltpu.VMEM((TPW, D_MODEL), jnp.float32),
            pltpu.SemaphoreType.DMA,
        ],
    )
    out = sc_fn(lgt, pe_flat, gen_full)
    return (out, jnp.float32(0.0))
